# Initial kernel scaffold; baseline (speedup 1.0000x reference)
import jax
import jax.numpy as jnp
from jax.experimental import pallas as pl


def kernel(x):
    raise NotImplementedError("write your pallas kernel here")


# speedup vs baseline: 1.4348x; 1.4348x over previous
import jax
import jax.numpy as jnp
import numpy as np
from jax.experimental import pallas as pl
from jax.experimental.pallas import tpu as pltpu

PATCH_L = 512
PATCH_C = 64
NUM_PATCHES = 16
F_S = 100.0

_N = PATCH_L
_kn = (np.arange(_N)[:, None] * np.arange(_N)[None, :]) % _N
_ang = 2.0 * np.pi * _kn / _N
_DFT = np.concatenate([np.cos(_ang), -np.sin(_ang)], axis=0).astype(np.float32)


def _patch_starts(B, L, C):
    kL, kC = jax.random.split(jax.random.key(42))
    start_L = jax.random.randint(kL, (B, NUM_PATCHES), 0, L - PATCH_L + 1)
    start_C = jax.random.randint(kC, (B, NUM_PATCHES), 0, C - PATCH_C + 1)
    return start_L, start_C


def _fft_body(sl_ref, sc_ref, x_ref, dft_ref, ore_ref, oim_ref):
    b = pl.program_id(0)
    p = pl.program_id(1)
    i = b * NUM_PATCHES + p
    sl = sl_ref[i]
    sc = sc_ref[i]
    C = x_ref.shape[2]
    xs = x_ref[0, pl.ds(sl, PATCH_L), :]
    c_iota = jax.lax.broadcasted_iota(jnp.int32, (C, PATCH_C), 0)
    j_iota = jax.lax.broadcasted_iota(jnp.int32, (C, PATCH_C), 1)
    sel = (c_iota == j_iota + sc).astype(jnp.float32)
    patch = jax.lax.dot_general(
        xs, sel, (((1,), (0,)), ((), ())),
        preferred_element_type=jnp.float32,
        precision=jax.lax.Precision.HIGHEST,
    )
    res = jax.lax.dot_general(
        dft_ref[...], patch, (((1,), (0,)), ((), ())),
        preferred_element_type=jnp.float32,
        precision=jax.lax.Precision.HIGHEST,
    )
    ore_ref[0, 0] = res[:PATCH_L]
    oim_ref[0, 0] = res[PATCH_L:]


def kernel(x):
    B, L, C = x.shape
    start_L, start_C = _patch_starts(B, L, C)
    sl_flat = start_L.reshape(-1).astype(jnp.int32)
    sc_flat = start_C.reshape(-1).astype(jnp.int32)
    dft = jnp.asarray(_DFT)

    grid_spec = pltpu.PrefetchScalarGridSpec(
        num_scalar_prefetch=2,
        grid=(B, NUM_PATCHES),
        in_specs=[
            pl.BlockSpec((1, L, C), lambda b, p, *_: (b, 0, 0)),
            pl.BlockSpec((2 * PATCH_L, PATCH_L), lambda b, p, *_: (0, 0)),
        ],
        out_specs=[
            pl.BlockSpec((1, 1, PATCH_L, PATCH_C), lambda b, p, *_: (b, p, 0, 0)),
            pl.BlockSpec((1, 1, PATCH_L, PATCH_C), lambda b, p, *_: (b, p, 0, 0)),
        ],
    )
    ore, oim = pl.pallas_call(
        _fft_body,
        grid_spec=grid_spec,
        out_shape=[
            jax.ShapeDtypeStruct((B, NUM_PATCHES, PATCH_L, PATCH_C), jnp.float32),
            jax.ShapeDtypeStruct((B, NUM_PATCHES, PATCH_L, PATCH_C), jnp.float32),
        ],
    )(sl_flat, sc_flat, x, dft)

    patches_fft = jnp.stack([ore, oim], axis=-1)
    t = jnp.broadcast_to(
        (jnp.arange(L, dtype=jnp.float32) * (1.0 / F_S))[None, :], (B, L)
    )
    return (patches_fft, t)


# speedup vs baseline: 2.5010x; 1.7431x over previous
import jax
import jax.numpy as jnp
import numpy as np
from jax.experimental import pallas as pl
from jax.experimental.pallas import tpu as pltpu

PATCH_L = 512
PATCH_C = 64
NUM_PATCHES = 16
F_S = 100.0

_N = PATCH_L
_kn = (np.arange(_N)[:, None] * np.arange(_N)[None, :]) % _N
_ang = 2.0 * np.pi * _kn / _N
_DFT = np.concatenate([np.cos(_ang), -np.sin(_ang)], axis=0).astype(np.float32)


def _patch_starts(B, L, C):
    kL, kC = jax.random.split(jax.random.key(42))
    start_L = jax.random.randint(kL, (B, NUM_PATCHES), 0, L - PATCH_L + 1)
    start_C = jax.random.randint(kC, (B, NUM_PATCHES), 0, C - PATCH_C + 1)
    return start_L, start_C


def _fft_body(sl_ref, sc_ref, x_ref, dft_ref, ore_ref, oim_ref):
    b = pl.program_id(0)
    p = pl.program_id(1)
    i = b * NUM_PATCHES + p
    sl = sl_ref[i]
    sc = sc_ref[i]
    C = x_ref.shape[2]
    xs = x_ref[0, pl.ds(sl, PATCH_L), :]
    c_iota = jax.lax.broadcasted_iota(jnp.int32, (C, PATCH_C), 0)
    j_iota = jax.lax.broadcasted_iota(jnp.int32, (C, PATCH_C), 1)
    sel = (c_iota == j_iota + sc).astype(jnp.float32)
    patch = jax.lax.dot_general(
        xs, sel, (((1,), (0,)), ((), ())),
        preferred_element_type=jnp.float32,
        precision=jax.lax.Precision.HIGHEST,
    )
    res = jax.lax.dot_general(
        dft_ref[...], patch, (((1,), (0,)), ((), ())),
        preferred_element_type=jnp.float32,
    )
    ore_ref[0, 0] = res[:PATCH_L]
    oim_ref[0, 0] = res[PATCH_L:]


def kernel(x):
    B, L, C = x.shape
    start_L, start_C = _patch_starts(B, L, C)
    sl_flat = start_L.reshape(-1).astype(jnp.int32)
    sc_flat = start_C.reshape(-1).astype(jnp.int32)
    dft = jnp.asarray(_DFT)

    grid_spec = pltpu.PrefetchScalarGridSpec(
        num_scalar_prefetch=2,
        grid=(B, NUM_PATCHES),
        in_specs=[
            pl.BlockSpec((1, L, C), lambda b, p, *_: (b, 0, 0)),
            pl.BlockSpec((2 * PATCH_L, PATCH_L), lambda b, p, *_: (0, 0)),
        ],
        out_specs=[
            pl.BlockSpec((1, 1, PATCH_L, PATCH_C), lambda b, p, *_: (b, p, 0, 0)),
            pl.BlockSpec((1, 1, PATCH_L, PATCH_C), lambda b, p, *_: (b, p, 0, 0)),
        ],
    )
    ore, oim = pl.pallas_call(
        _fft_body,
        grid_spec=grid_spec,
        out_shape=[
            jax.ShapeDtypeStruct((B, NUM_PATCHES, PATCH_L, PATCH_C), jnp.float32),
            jax.ShapeDtypeStruct((B, NUM_PATCHES, PATCH_L, PATCH_C), jnp.float32),
        ],
    )(sl_flat, sc_flat, x, dft)

    patches_fft = jnp.stack([ore, oim], axis=-1)
    t = jnp.broadcast_to(
        (jnp.arange(L, dtype=jnp.float32) * (1.0 / F_S))[None, :], (B, L)
    )
    return (patches_fft, t)


# speedup vs baseline: 5.5502x; 2.2191x over previous
import jax
import jax.numpy as jnp
import numpy as np
from jax.experimental import pallas as pl
from jax.experimental.pallas import tpu as pltpu

PATCH_L = 512
PATCH_C = 64
NUM_PATCHES = 16
F_S = 100.0

_N = PATCH_L
_kn = (np.arange(_N)[:, None] * np.arange(_N)[None, :]) % _N
_ang = 2.0 * np.pi * _kn / _N
_DFT = np.concatenate([np.cos(_ang), -np.sin(_ang)], axis=0).astype(np.float32)


def _patch_starts(B, L, C):
    kL, kC = jax.random.split(jax.random.key(42))
    start_L = jax.random.randint(kL, (B, NUM_PATCHES), 0, L - PATCH_L + 1)
    start_C = jax.random.randint(kC, (B, NUM_PATCHES), 0, C - PATCH_C + 1)
    return start_L, start_C


def _fft_body(sl_ref, sc_ref, x_ref, dft_ref, ore_ref, oim_ref):
    b = pl.program_id(0)
    C = x_ref.shape[2]
    c_iota = jax.lax.broadcasted_iota(jnp.int32, (C, PATCH_C), 0)
    j_iota = jax.lax.broadcasted_iota(jnp.int32, (C, PATCH_C), 1)
    cols = []
    for p in range(NUM_PATCHES):
        i = b * NUM_PATCHES + p
        sl = sl_ref[i]
        sc = sc_ref[i]
        xs = x_ref[0, pl.ds(sl, PATCH_L), :]
        sel = (c_iota == j_iota + sc).astype(jnp.float32)
        cols.append(jax.lax.dot_general(
            xs, sel, (((1,), (0,)), ((), ())),
            preferred_element_type=jnp.float32,
        ))
    patches = jnp.concatenate(cols, axis=1)
    res = jax.lax.dot_general(
        dft_ref[...], patches, (((1,), (0,)), ((), ())),
        preferred_element_type=jnp.float32,
    )
    for p in range(NUM_PATCHES):
        ore_ref[0, p] = res[:PATCH_L, p * PATCH_C:(p + 1) * PATCH_C]
        oim_ref[0, p] = res[PATCH_L:, p * PATCH_C:(p + 1) * PATCH_C]


def kernel(x):
    B, L, C = x.shape
    start_L, start_C = _patch_starts(B, L, C)
    sl_flat = start_L.reshape(-1).astype(jnp.int32)
    sc_flat = start_C.reshape(-1).astype(jnp.int32)
    dft = jnp.asarray(_DFT)

    grid_spec = pltpu.PrefetchScalarGridSpec(
        num_scalar_prefetch=2,
        grid=(B,),
        in_specs=[
            pl.BlockSpec((1, L, C), lambda b, *_: (b, 0, 0)),
            pl.BlockSpec((2 * PATCH_L, PATCH_L), lambda b, *_: (0, 0)),
        ],
        out_specs=[
            pl.BlockSpec((1, NUM_PATCHES, PATCH_L, PATCH_C),
                         lambda b, *_: (b, 0, 0, 0)),
            pl.BlockSpec((1, NUM_PATCHES, PATCH_L, PATCH_C),
                         lambda b, *_: (b, 0, 0, 0)),
        ],
    )
    ore, oim = pl.pallas_call(
        _fft_body,
        grid_spec=grid_spec,
        out_shape=[
            jax.ShapeDtypeStruct((B, NUM_PATCHES, PATCH_L, PATCH_C), jnp.float32),
            jax.ShapeDtypeStruct((B, NUM_PATCHES, PATCH_L, PATCH_C), jnp.float32),
        ],
    )(sl_flat, sc_flat, x, dft)

    patches_fft = jnp.stack([ore, oim], axis=-1)
    t = jnp.broadcast_to(
        (jnp.arange(L, dtype=jnp.float32) * (1.0 / F_S))[None, :], (B, L)
    )
    return (patches_fft, t)
